# Initial kernel scaffold; baseline (speedup 1.0000x reference)
import jax
import jax.numpy as jnp
from jax.experimental import pallas as pl


def kernel(pred_p3, pred_p4, pred_p5, boxes, labels, kpts):
    raise NotImplementedError("write your pallas kernel here")


# speedup vs baseline: 63.6661x; 63.6661x over previous
import jax
import jax.numpy as jnp
from jax import lax
from jax.experimental import pallas as pl
from jax.experimental.pallas import tpu as pltpu

_NC = 1
_NK = 17
_NO = _NC + 1 + _NK * 3
_N = 8
_LEVELS = ((4096, 64, 8.0), (1024, 32, 16.0), (256, 16, 32.0))


def _softplus(x):
    return jnp.maximum(x, 0.0) + jnp.log1p(jnp.exp(-jnp.abs(x)))


def _bce(x, t):
    return jnp.maximum(x, 0.0) - x * t + jnp.log1p(jnp.exp(-jnp.abs(x)))


def _smooth_l1(d):
    ad = jnp.abs(d)
    return jnp.where(ad < 1.0, 0.5 * d * d, ad - 0.5)


def _sel(col_expr):
    r = lax.broadcasted_iota(jnp.int32, (_NO, _NK), 0)
    k = lax.broadcasted_iota(jnp.int32, (_NO, _NK), 1)
    return (r == col_expr(k)).astype(jnp.float32)


def _loss_body(p3_ref, p4_ref, p5_ref, boxes_ref, boxesT_ref,
               kx_ref, ky_ref, ks_ref,
               tot_ref, lo_ref, lc_ref, lk_ref):
    b = pl.program_id(0)

    @pl.when(b == 0)
    def _():
        tot_ref[0] = 0.0
        lo_ref[0] = 0.0
        lc_ref[0] = 0.0
        lk_ref[0] = 0.0

    bxy = boxes_ref[0]
    bxyT = boxesT_ref[0]
    kx = kx_ref[0]
    ky = ky_ref[0]
    ks = ks_ref[0]
    vis = (ks > 0.0).astype(jnp.float32)

    sel_x = _sel(lambda k: 3 * k + 2)
    sel_y = _sel(lambda k: 3 * k + 3)
    sel_s = _sel(lambda k: 3 * k + 4)

    lo_b = jnp.float32(0.0)
    lk_b = jnp.float32(0.0)
    for p_ref, (hw, w, stride) in zip((p3_ref, p4_ref, p5_ref), _LEVELS):
        p = p_ref[0]
        obj_row = p[_NC:_NC + 1, :]
        lo_b += jnp.sum(_softplus(obj_row)) / hw

        scale = 512.0 / stride
        gx = jnp.floor(bxy[:, 0:1] * scale).astype(jnp.int32)
        gy = jnp.floor(bxy[:, 1:2] * scale).astype(jnp.int32)
        cell = gy * w + gx
        gxT = jnp.floor(bxyT[0:1, :] * scale).astype(jnp.int32)
        gyT = jnp.floor(bxyT[1:2, :] * scale).astype(jnp.int32)
        cellT = gyT * w + gxT

        valid = jnp.logical_and(cell >= 0, cell < hw)
        cellc = jnp.clip(cell, 0, hw - 1)
        cellcT = jnp.clip(cellT, 0, hw - 1)
        validT = jnp.logical_and(cellT >= 0, cellT < hw)

        iota = lax.broadcasted_iota(jnp.int32, (_N, hw), 1)
        onehot = jnp.logical_and(iota == cellc, valid).astype(jnp.float32)
        g = lax.dot_general(onehot, p, (((1,), (1,)), ((), ())),
                            preferred_element_type=jnp.float32)

        ii = lax.broadcasted_iota(jnp.int32, (_N, _N), 0)
        jj = lax.broadcasted_iota(jnp.int32, (_N, _N), 1)
        same = jnp.logical_and(cellc == cellcT, jnp.logical_and(jj > ii, validT))
        overwritten = jnp.sum(same.astype(jnp.float32), axis=1, keepdims=True)
        lastw = jnp.where(jnp.logical_and(overwritten == 0.0, valid), 1.0, 0.0)

        lo_b -= jnp.sum(lastw * g[:, _NC:_NC + 1]) / hw

        kpx = lax.dot_general(g, sel_x, (((1,), (0,)), ((), ())),
                              preferred_element_type=jnp.float32)
        kpy = lax.dot_general(g, sel_y, (((1,), (0,)), ((), ())),
                              preferred_element_type=jnp.float32)
        kps = lax.dot_general(g, sel_s, (((1,), (0,)), ((), ())),
                              preferred_element_type=jnp.float32)
        m = lastw * vis
        lxy = (_smooth_l1(kpx - kx) + _smooth_l1(kpy - ky)) * m
        lsc = _bce(kps, ks) * m
        den = jnp.sum(m) + 1e-6
        lk_b += jnp.sum(lxy) / den + jnp.sum(lsc) / den

    lo_ref[0] += lo_b
    lk_ref[0] += lk_b
    tot_ref[0] += 90.0 * (lo_b + lk_b)


def kernel(pred_p3, pred_p4, pred_p5, boxes, labels, kpts):
    del labels
    B = pred_p3.shape[0]
    p3 = pred_p3.reshape(B, _NO, 4096)
    p4 = pred_p4.reshape(B, _NO, 1024)
    p5 = pred_p5.reshape(B, _NO, 256)
    boxesT = jnp.transpose(boxes, (0, 2, 1))
    kx = kpts[:, :, :, 0]
    ky = kpts[:, :, :, 1]
    ks = kpts[:, :, :, 2]

    smem_out = pl.BlockSpec(memory_space=pltpu.MemorySpace.SMEM)
    outs = pl.pallas_call(
        _loss_body,
        grid=(B,),
        in_specs=[
            pl.BlockSpec((1, _NO, 4096), lambda b: (b, 0, 0)),
            pl.BlockSpec((1, _NO, 1024), lambda b: (b, 0, 0)),
            pl.BlockSpec((1, _NO, 256), lambda b: (b, 0, 0)),
            pl.BlockSpec((1, _N, 4), lambda b: (b, 0, 0)),
            pl.BlockSpec((1, 4, _N), lambda b: (b, 0, 0)),
            pl.BlockSpec((1, _N, _NK), lambda b: (b, 0, 0)),
            pl.BlockSpec((1, _N, _NK), lambda b: (b, 0, 0)),
            pl.BlockSpec((1, _N, _NK), lambda b: (b, 0, 0)),
        ],
        out_specs=[smem_out, smem_out, smem_out, smem_out],
        out_shape=[jax.ShapeDtypeStruct((1,), jnp.float32)] * 4,
    )(p3, p4, p5, boxes, boxesT, kx, ky, ks)
    tot, lo, lc, lk = outs
    return tot[0], lo[0], lc[0], lk[0]


# speedup vs baseline: 76.0154x; 1.1940x over previous
import jax
import jax.numpy as jnp
from jax import lax
from jax.experimental import pallas as pl
from jax.experimental.pallas import tpu as pltpu

_NC = 1
_NK = 17
_NO = _NC + 1 + _NK * 3
_N = 8
_B = 16
_LEVELS = ((4096, 64, 8.0), (1024, 32, 16.0), (256, 16, 32.0))


def _softplus(x):
    return jnp.maximum(x, 0.0) + jnp.log1p(jnp.exp(-jnp.abs(x)))


def _bce(x, t):
    return jnp.maximum(x, 0.0) - x * t + jnp.log1p(jnp.exp(-jnp.abs(x)))


def _smooth_l1(d):
    ad = jnp.abs(d)
    return jnp.where(ad < 1.0, 0.5 * d * d, ad - 0.5)


def _sel(col_expr):
    r = lax.broadcasted_iota(jnp.int32, (_NO, _NK), 0)
    k = lax.broadcasted_iota(jnp.int32, (_NO, _NK), 1)
    return (r == col_expr(k)).astype(jnp.float32)


def _loss_body(p3_ref, p4_ref, p5_ref, boxes_ref, boxesT_ref,
               kx_ref, ky_ref, ks_ref,
               tot_ref, lo_ref, lc_ref, lk_ref):
    vis = (ks_ref[...] > 0.0).astype(jnp.float32)
    kx = kx_ref[...]
    ky = ky_ref[...]
    ks = ks_ref[...]
    bxy = boxes_ref[...]
    bxyT = boxesT_ref[...]

    sel_x = _sel(lambda k: 3 * k + 2)
    sel_y = _sel(lambda k: 3 * k + 3)
    sel_s = _sel(lambda k: 3 * k + 4)

    lo = jnp.float32(0.0)
    lk = jnp.float32(0.0)
    for p_ref, (hw, w, stride) in zip((p3_ref, p4_ref, p5_ref), _LEVELS):
        p = p_ref[...]
        lo += jnp.sum(_softplus(p[:, _NC:_NC + 1, :])) / hw

        scale = 512.0 / stride
        gx = jnp.floor(bxy[:, :, 0:1] * scale).astype(jnp.int32)
        gy = jnp.floor(bxy[:, :, 1:2] * scale).astype(jnp.int32)
        cell = gy * w + gx
        gxT = jnp.floor(bxyT[:, 0:1, :] * scale).astype(jnp.int32)
        gyT = jnp.floor(bxyT[:, 1:2, :] * scale).astype(jnp.int32)
        cellT = gyT * w + gxT

        valid = jnp.logical_and(cell >= 0, cell < hw)
        cellc = jnp.clip(cell, 0, hw - 1)
        validT = jnp.logical_and(cellT >= 0, cellT < hw)
        cellcT = jnp.clip(cellT, 0, hw - 1)

        iota = lax.broadcasted_iota(jnp.int32, (_B, _N, hw), 2)
        onehot = jnp.logical_and(iota == cellc, valid).astype(jnp.float32)
        g = lax.dot_general(onehot, p, (((2,), (2,)), ((0,), (0,))),
                            preferred_element_type=jnp.float32)

        ii = lax.broadcasted_iota(jnp.int32, (_B, _N, _N), 1)
        jj = lax.broadcasted_iota(jnp.int32, (_B, _N, _N), 2)
        same = jnp.logical_and(cellc == cellcT,
                               jnp.logical_and(jj > ii, validT))
        overwritten = jnp.sum(same.astype(jnp.float32), axis=2, keepdims=True)
        lastw = jnp.where(jnp.logical_and(overwritten == 0.0, valid), 1.0, 0.0)

        lo -= jnp.sum(lastw * g[:, :, _NC:_NC + 1]) / hw

        kpx = lax.dot_general(g, sel_x, (((2,), (0,)), ((), ())),
                              preferred_element_type=jnp.float32)
        kpy = lax.dot_general(g, sel_y, (((2,), (0,)), ((), ())),
                              preferred_element_type=jnp.float32)
        kps = lax.dot_general(g, sel_s, (((2,), (0,)), ((), ())),
                              preferred_element_type=jnp.float32)
        m = lastw * vis
        lxy = (_smooth_l1(kpx - kx) + _smooth_l1(kpy - ky)) * m
        lsc = _bce(kps, ks) * m
        den = jnp.sum(m, axis=(1, 2), keepdims=True) + 1e-6
        lk += jnp.sum(jnp.sum(lxy, axis=(1, 2), keepdims=True) / den)
        lk += jnp.sum(jnp.sum(lsc, axis=(1, 2), keepdims=True) / den)

    lo_ref[0] = lo
    lc_ref[0] = 0.0
    lk_ref[0] = lk
    tot_ref[0] = 90.0 * (lo + lk)


def kernel(pred_p3, pred_p4, pred_p5, boxes, labels, kpts):
    del labels
    p3 = pred_p3.reshape(_B, _NO, 4096)
    p4 = pred_p4.reshape(_B, _NO, 1024)
    p5 = pred_p5.reshape(_B, _NO, 256)
    boxesT = jnp.transpose(boxes, (0, 2, 1))
    kx = kpts[:, :, :, 0]
    ky = kpts[:, :, :, 1]
    ks = kpts[:, :, :, 2]

    smem_out = pl.BlockSpec(memory_space=pltpu.MemorySpace.SMEM)
    outs = pl.pallas_call(
        _loss_body,
        out_specs=[smem_out, smem_out, smem_out, smem_out],
        out_shape=[jax.ShapeDtypeStruct((1,), jnp.float32)] * 4,
    )(p3, p4, p5, boxes, boxesT, kx, ky, ks)
    tot, lo, lc, lk = outs
    return tot[0], lo[0], lc[0], lk[0]
